# baseline (speedup 1.0000x reference)
import jax
import jax.numpy as jnp
import numpy as np
from jax.experimental import pallas as pl
from jax.experimental.pallas import tpu as pltpu

B, S, DIM, QLR = 1, 2048, 2048, 1536
H, HD, RD, TOPK = 16, 128, 64, 512
BQ = 256
NBQ = S // BQ
CH = 512
SC_HD = np.float32(HD ** -0.5)


def _hadamard(n):
    Hm = np.array([[1.0]], dtype=np.float32)
    while Hm.shape[0] < n:
        Hm = np.block([[Hm, Hm], [Hm, -Hm]])
    return Hm


_HMAT_NP = _hadamard(HD)


def _apply_rotary(x, cos, sin):
    xr = x[..., 0::2]
    xi = x[..., 1::2]
    c = cos[None, :, None, :]
    s = sin[None, :, None, :]
    o_r = xr * c - xi * s
    o_i = xr * s + xi * c
    return jnp.stack([o_r, o_i], axis=-1).reshape(x.shape)


def _layernorm(x, w, b, eps=1e-6):
    xf = x.astype(jnp.float32)
    m = jnp.mean(xf, axis=-1, keepdims=True)
    v = jnp.var(xf, axis=-1, keepdims=True)
    return ((xf - m) / jnp.sqrt(v + eps) * w + b).astype(x.dtype)


def _had_kernel(q_ref, k_ref, hmat_ref, q_out, k_out):
    hmat = hmat_ref[...]
    qb = q_ref[...].astype(jnp.bfloat16)
    qh = jnp.concatenate(
        [jnp.dot(qb[:, h * HD:(h + 1) * HD], hmat,
                 preferred_element_type=jnp.float32) for h in range(H)],
        axis=1) * SC_HD
    q_out[...] = qh.astype(jnp.bfloat16)
    kh = jnp.dot(k_ref[...].astype(jnp.bfloat16), hmat,
                 preferred_element_type=jnp.float32) * SC_HD
    k_out[...] = kh.astype(jnp.bfloat16)


def _col_iota(w):
    return jax.lax.broadcasted_iota(jnp.int32, (1, w), 1)


def _partner(v, j):
    w = v.shape[1]
    if j >= 128:
        g = j // 128
        ng = w // 128
        parts = [v[:, (c ^ g) * 128:((c ^ g) + 1) * 128] for c in range(ng)]
        return jnp.concatenate(parts, axis=1)
    sel_low = (_col_iota(w) & j) == 0
    return jnp.where(sel_low, jnp.roll(v, -j, axis=1), jnp.roll(v, j, axis=1))


def _cmpex(key, idx, j, desc):
    sel_low = (_col_iota(key.shape[1]) & j) == 0
    pk = _partner(key, j)
    pi = _partner(idx, j)
    take = (key > pk) == (sel_low == desc)
    return jnp.where(take, key, pk), jnp.where(take, idx, pi)


def _sort_chunks(key, idx, base_desc_fn):
    col = _col_iota(key.shape[1])
    base = base_desc_fn(col)
    k = 2
    while k <= CH:
        j = k // 2
        while j >= 1:
            desc = jnp.logical_xor(((col % CH) & k) != 0, base)
            key, idx = _cmpex(key, idx, j, desc)
            j //= 2
        k *= 2
    return key, idx


def _merge_pairs(key, idx, mc):
    pairs, rem = mc // 2, mc % 2
    kparts, iparts = [], []
    for c in range(pairs):
        a = slice(2 * c * CH, 2 * c * CH + CH)
        b = slice(2 * c * CH + CH, 2 * c * CH + 2 * CH)
        ka, kb = key[:, a], key[:, b]
        ia, ib = idx[:, a], idx[:, b]
        cond = ka > kb
        kparts.append(jnp.where(cond, ka, kb))
        iparts.append(jnp.where(cond, ia, ib))
    if rem:
        kparts.append(key[:, 2 * pairs * CH:])
        iparts.append(idx[:, 2 * pairs * CH:])
    if len(kparts) > 1:
        return jnp.concatenate(kparts, axis=1), jnp.concatenate(iparts, axis=1)
    return kparts[0], iparts[0]


def _merge_sort(key, idx, base_desc_fn):
    col = _col_iota(key.shape[1])
    base = base_desc_fn(col)
    j = CH // 2
    while j >= 1:
        key, idx = _cmpex(key, idx, j, base)
        j //= 2
    return key, idx


def _make_score_topk_kernel(g, nc):
    W = nc * CH

    if nc == 1:
        p1 = lambda col: jnp.full(col.shape, True)
    elif nc == 3:
        p1 = lambda col: col < CH
    else:
        p1 = lambda col: (col & CH) == 0

    def kern(q_ref, k_ref, w_ref, vals_ref, idx_ref):
        qi = 2 * g + pl.program_id(0)
        kmat = k_ref[...]
        w = w_ref[...]
        acc = jnp.zeros((BQ, W), dtype=jnp.float32)
        for h in range(H):
            lg = jax.lax.dot_general(q_ref[:, h * HD:(h + 1) * HD], kmat,
                                     (((1,), (1,)), ((), ())),
                                     preferred_element_type=jnp.float32)
            acc = acc + jnp.maximum(lg, 0.0) * w[:, h:h + 1]
        row = qi * BQ + jax.lax.broadcasted_iota(jnp.int32, (BQ, W), 0)
        col = _col_iota(W)
        key = jnp.where(col > row, -(1e9 + 64.0 * col.astype(jnp.float32)), acc)
        idx = jnp.broadcast_to(col, (BQ, W))

        key, idx = _sort_chunks(key, idx, p1)
        mc = nc
        while mc > 1:
            key, idx = _merge_pairs(key, idx, mc)
            mc = mc // 2 + mc % 2
            if mc == 1:
                d = lambda col: jnp.full(col.shape, True)
            else:
                d = lambda col: (col & CH) == 0
            key, idx = _merge_sort(key, idx, d)

        vals_ref[...] = jnp.where(key < -1e8, jnp.float32(-1e9), key)
        idx_ref[...] = idx

    return kern


def kernel(x, qr, Wq_b, Wk, knorm_w, knorm_b, Wweights, freqs_cos, freqs_sin, mask):
    q = jnp.matmul(qr, Wq_b.T).reshape(B, S, H, HD)
    q_pe = _apply_rotary(q[..., :RD], freqs_cos, freqs_sin)
    q = jnp.concatenate([q_pe, q[..., RD:]], axis=-1)
    k = jnp.matmul(x, Wk.T)
    k = _layernorm(k, knorm_w, knorm_b)
    k_pe = _apply_rotary(k[..., :RD][:, :, None, :], freqs_cos, freqs_sin)[:, :, 0, :]
    k = jnp.concatenate([k_pe, k[..., RD:]], axis=-1)
    weights = jnp.matmul(x, Wweights.T) * (H ** -0.5)
    wmat = weights[0] * SC_HD

    q2 = q.reshape(S, H * HD)
    k2 = k[0]

    qmat, kmat = pl.pallas_call(
        _had_kernel,
        grid=(NBQ,),
        in_specs=[
            pl.BlockSpec((BQ, H * HD), lambda i: (i, 0)),
            pl.BlockSpec((BQ, HD), lambda i: (i, 0)),
            pl.BlockSpec((HD, HD), lambda i: (0, 0)),
        ],
        out_specs=[
            pl.BlockSpec((BQ, H * HD), lambda i: (i, 0)),
            pl.BlockSpec((BQ, HD), lambda i: (i, 0)),
        ],
        out_shape=[
            jax.ShapeDtypeStruct((S, H * HD), jnp.bfloat16),
            jax.ShapeDtypeStruct((S, HD), jnp.bfloat16),
        ],
        compiler_params=pltpu.CompilerParams(
            dimension_semantics=("parallel",)),
    )(q2, k2, jnp.asarray(_HMAT_NP, dtype=jnp.bfloat16))

    vparts, iparts = [], []
    for g in range(4):
        nc = (2 * g + 2 + 1) // 2
        W = nc * CH
        v, i = pl.pallas_call(
            _make_score_topk_kernel(g, nc),
            grid=(2,),
            in_specs=[
                pl.BlockSpec((BQ, H * HD), lambda i, g=g: (2 * g + i, 0)),
                pl.BlockSpec((W, HD), lambda i: (0, 0)),
                pl.BlockSpec((BQ, H), lambda i, g=g: (2 * g + i, 0)),
            ],
            out_specs=[
                pl.BlockSpec((BQ, TOPK), lambda i: (i, 0)),
                pl.BlockSpec((BQ, TOPK), lambda i: (i, 0)),
            ],
            out_shape=[
                jax.ShapeDtypeStruct((2 * BQ, TOPK), jnp.float32),
                jax.ShapeDtypeStruct((2 * BQ, TOPK), jnp.int32),
            ],
            compiler_params=pltpu.CompilerParams(
                dimension_semantics=("parallel",)),
        )(qmat, kmat, wmat)
        vparts.append(v)
        iparts.append(i)

    vals = jnp.concatenate(vparts, axis=0)[None]
    idx = jnp.concatenate(iparts, axis=0)[None]
    return vals, idx


# speedup vs baseline: 1.0007x; 1.0007x over previous
import jax
import jax.numpy as jnp
import numpy as np
from jax.experimental import pallas as pl
from jax.experimental.pallas import tpu as pltpu

B, S, DIM, QLR = 1, 2048, 2048, 1536
H, HD, RD, TOPK = 16, 128, 64, 512
BQ = 256
NBQ = S // BQ
CH = 512
SC_HD = np.float32(HD ** -0.5)


def _hadamard(n):
    Hm = np.array([[1.0]], dtype=np.float32)
    while Hm.shape[0] < n:
        Hm = np.block([[Hm, Hm], [Hm, -Hm]])
    return Hm


_HMAT_NP = _hadamard(HD)


def _apply_rotary(x, cos, sin):
    xr = x[..., 0::2]
    xi = x[..., 1::2]
    c = cos[None, :, None, :]
    s = sin[None, :, None, :]
    o_r = xr * c - xi * s
    o_i = xr * s + xi * c
    return jnp.stack([o_r, o_i], axis=-1).reshape(x.shape)


def _layernorm(x, w, b, eps=1e-6):
    xf = x.astype(jnp.float32)
    m = jnp.mean(xf, axis=-1, keepdims=True)
    v = jnp.var(xf, axis=-1, keepdims=True)
    return ((xf - m) / jnp.sqrt(v + eps) * w + b).astype(x.dtype)


def _had_kernel(q_ref, k_ref, hmat_ref, q_out, k_out):
    hmat = hmat_ref[...]
    qb = q_ref[...].astype(jnp.bfloat16)
    qh = jnp.concatenate(
        [jnp.dot(qb[:, h * HD:(h + 1) * HD], hmat,
                 preferred_element_type=jnp.float32) for h in range(H)],
        axis=1) * SC_HD
    q_out[...] = qh.astype(jnp.bfloat16)
    kh = jnp.dot(k_ref[...].astype(jnp.bfloat16), hmat,
                 preferred_element_type=jnp.float32) * SC_HD
    k_out[...] = kh.astype(jnp.bfloat16)


def _col_iota(w):
    return jax.lax.broadcasted_iota(jnp.int32, (1, w), 1)


def _partner(v, j):
    w = v.shape[1]
    if j >= 128:
        g = j // 128
        ng = w // 128
        parts = [v[:, (c ^ g) * 128:((c ^ g) + 1) * 128] for c in range(ng)]
        return jnp.concatenate(parts, axis=1)
    sel_low = (_col_iota(w) & j) == 0
    left = jnp.concatenate([v[:, j:], v[:, :j]], axis=1)
    right = jnp.concatenate([v[:, w - j:], v[:, :w - j]], axis=1)
    return jnp.where(sel_low, left, right)


def _cmpex(key, idx, j, desc):
    sel_low = (_col_iota(key.shape[1]) & j) == 0
    pk = _partner(key, j)
    pi = _partner(idx, j)
    take = (key > pk) == (sel_low == desc)
    return jnp.where(take, key, pk), jnp.where(take, idx, pi)


def _sort_chunks(key, idx, base_desc_fn):
    col = _col_iota(key.shape[1])
    base = base_desc_fn(col)
    k = 2
    while k <= CH:
        j = k // 2
        while j >= 1:
            desc = jnp.logical_xor(((col % CH) & k) != 0, base)
            key, idx = _cmpex(key, idx, j, desc)
            j //= 2
        k *= 2
    return key, idx


def _merge_pairs(key, idx, mc):
    pairs, rem = mc // 2, mc % 2
    kparts, iparts = [], []
    for c in range(pairs):
        a = slice(2 * c * CH, 2 * c * CH + CH)
        b = slice(2 * c * CH + CH, 2 * c * CH + 2 * CH)
        ka, kb = key[:, a], key[:, b]
        ia, ib = idx[:, a], idx[:, b]
        cond = ka > kb
        kparts.append(jnp.where(cond, ka, kb))
        iparts.append(jnp.where(cond, ia, ib))
    if rem:
        kparts.append(key[:, 2 * pairs * CH:])
        iparts.append(idx[:, 2 * pairs * CH:])
    if len(kparts) > 1:
        return jnp.concatenate(kparts, axis=1), jnp.concatenate(iparts, axis=1)
    return kparts[0], iparts[0]


def _merge_sort(key, idx, base_desc_fn):
    col = _col_iota(key.shape[1])
    base = base_desc_fn(col)
    j = CH // 2
    while j >= 1:
        key, idx = _cmpex(key, idx, j, base)
        j //= 2
    return key, idx


def _make_score_topk_kernel(g, nc):
    W = nc * CH

    if nc == 1:
        p1 = lambda col: jnp.full(col.shape, True)
    elif nc == 3:
        p1 = lambda col: col < CH
    else:
        p1 = lambda col: (col & CH) == 0

    def kern(q_ref, k_ref, w_ref, vals_ref, idx_ref):
        qi = 2 * g + pl.program_id(0)
        kmat = k_ref[...]
        w = w_ref[...]
        acc = jnp.zeros((BQ, W), dtype=jnp.float32)
        for h in range(H):
            lg = jax.lax.dot_general(q_ref[:, h * HD:(h + 1) * HD], kmat,
                                     (((1,), (1,)), ((), ())),
                                     preferred_element_type=jnp.float32)
            acc = acc + jnp.maximum(lg, 0.0) * w[:, h:h + 1]
        row = qi * BQ + jax.lax.broadcasted_iota(jnp.int32, (BQ, W), 0)
        col = _col_iota(W)
        key = jnp.where(col > row, -(1e9 + 64.0 * col.astype(jnp.float32)), acc)
        idx = jnp.broadcast_to(col, (BQ, W))

        key, idx = _sort_chunks(key, idx, p1)
        mc = nc
        while mc > 1:
            key, idx = _merge_pairs(key, idx, mc)
            mc = mc // 2 + mc % 2
            if mc == 1:
                d = lambda col: jnp.full(col.shape, True)
            else:
                d = lambda col: (col & CH) == 0
            key, idx = _merge_sort(key, idx, d)

        vals_ref[...] = jnp.where(key < -1e8, jnp.float32(-1e9), key)
        idx_ref[...] = idx

    return kern


def kernel(x, qr, Wq_b, Wk, knorm_w, knorm_b, Wweights, freqs_cos, freqs_sin, mask):
    q = jnp.matmul(qr, Wq_b.T).reshape(B, S, H, HD)
    q_pe = _apply_rotary(q[..., :RD], freqs_cos, freqs_sin)
    q = jnp.concatenate([q_pe, q[..., RD:]], axis=-1)
    k = jnp.matmul(x, Wk.T)
    k = _layernorm(k, knorm_w, knorm_b)
    k_pe = _apply_rotary(k[..., :RD][:, :, None, :], freqs_cos, freqs_sin)[:, :, 0, :]
    k = jnp.concatenate([k_pe, k[..., RD:]], axis=-1)
    weights = jnp.matmul(x, Wweights.T) * (H ** -0.5)
    wmat = weights[0] * SC_HD

    q2 = q.reshape(S, H * HD)
    k2 = k[0]

    qmat, kmat = pl.pallas_call(
        _had_kernel,
        grid=(NBQ,),
        in_specs=[
            pl.BlockSpec((BQ, H * HD), lambda i: (i, 0)),
            pl.BlockSpec((BQ, HD), lambda i: (i, 0)),
            pl.BlockSpec((HD, HD), lambda i: (0, 0)),
        ],
        out_specs=[
            pl.BlockSpec((BQ, H * HD), lambda i: (i, 0)),
            pl.BlockSpec((BQ, HD), lambda i: (i, 0)),
        ],
        out_shape=[
            jax.ShapeDtypeStruct((S, H * HD), jnp.bfloat16),
            jax.ShapeDtypeStruct((S, HD), jnp.bfloat16),
        ],
        compiler_params=pltpu.CompilerParams(
            dimension_semantics=("parallel",)),
    )(q2, k2, jnp.asarray(_HMAT_NP, dtype=jnp.bfloat16))

    vparts, iparts = [], []
    for g in range(4):
        nc = (2 * g + 2 + 1) // 2
        W = nc * CH
        v, i = pl.pallas_call(
            _make_score_topk_kernel(g, nc),
            grid=(2,),
            in_specs=[
                pl.BlockSpec((BQ, H * HD), lambda i, g=g: (2 * g + i, 0)),
                pl.BlockSpec((W, HD), lambda i: (0, 0)),
                pl.BlockSpec((BQ, H), lambda i, g=g: (2 * g + i, 0)),
            ],
            out_specs=[
                pl.BlockSpec((BQ, TOPK), lambda i: (i, 0)),
                pl.BlockSpec((BQ, TOPK), lambda i: (i, 0)),
            ],
            out_shape=[
                jax.ShapeDtypeStruct((2 * BQ, TOPK), jnp.float32),
                jax.ShapeDtypeStruct((2 * BQ, TOPK), jnp.int32),
            ],
            compiler_params=pltpu.CompilerParams(
                dimension_semantics=("parallel",)),
        )(qmat, kmat, wmat)
        vparts.append(v)
        iparts.append(i)

    vals = jnp.concatenate(vparts, axis=0)[None]
    idx = jnp.concatenate(iparts, axis=0)[None]
    return vals, idx
